# Initial kernel scaffold; baseline (speedup 1.0000x reference)
import jax
import jax.numpy as jnp
from jax.experimental import pallas as pl


def kernel(x, Wg, bg, We, be, Wc1, bc1, Wc2, bc2, priority):
    raise NotImplementedError("write your pallas kernel here")


# speedup vs baseline: 4.0625x; 4.0625x over previous
import functools

import jax
import jax.numpy as jnp
from jax import lax
from jax.experimental import pallas as pl
from jax.experimental.pallas import tpu as pltpu
from jax.experimental.pallas import tpu_sc as plsc

_TC_TILE = 512
_STATS_W = 24


def _tc_stats_body(x_ref, w24_ref, b24_ref, p24_ref, wc1_ref, bc1_ref,
                   wc2_ref, bc2_ref, out_ref):
    xw = x_ref[...]
    dn = (((1,), (0,)), ((), ()))
    h = lax.dot_general(xw, wc1_ref[...], dn,
                        preferred_element_type=jnp.float32)
    h = h + bc1_ref[...]
    h = 0.5 * h * (1.0 + lax.erf(h * 0.7071067811865476))
    c24 = lax.dot_general(h.astype(jnp.bfloat16), wc2_ref[...], dn,
                          preferred_element_type=jnp.float32)
    conf = 1.0 / (1.0 + jnp.exp(-(c24 + bc2_ref[...])))
    lg = lax.dot_general(xw, w24_ref[...], dn,
                         preferred_element_type=jnp.float32)
    lg = (lg + b24_ref[...]) * p24_ref[...]
    lane = lax.broadcasted_iota(jnp.int32, lg.shape, 1)
    out_ref[...] = lg + jnp.where(lane == 20, conf, 0.0)


def _tc_stats(x_flat, w24, b24, p24, wc1, bc1, wc2, bc2):
    n, h = x_flat.shape
    t = _TC_TILE
    grid = n // t
    return pl.pallas_call(
        _tc_stats_body,
        grid=(grid,),
        in_specs=[
            pl.BlockSpec((t, h), lambda i: (i, 0)),
            pl.BlockSpec(w24.shape, lambda i: (0, 0)),
            pl.BlockSpec(b24.shape, lambda i: (0, 0)),
            pl.BlockSpec(p24.shape, lambda i: (0, 0)),
            pl.BlockSpec(wc1.shape, lambda i: (0, 0)),
            pl.BlockSpec(bc1.shape, lambda i: (0, 0)),
            pl.BlockSpec(wc2.shape, lambda i: (0, 0)),
            pl.BlockSpec(bc2.shape, lambda i: (0, 0)),
        ],
        out_specs=pl.BlockSpec((t, _STATS_W), lambda i: (i, 0)),
        out_shape=jax.ShapeDtypeStruct((n, _STATS_W), jnp.float32),
    )(x_flat, w24, b24, p24, wc1, bc1, wc2, bc2)


def _argmax_tb(vals, idxs, best_v, best_i):
    for v, i in zip(vals, idxs):
        take = (v > best_v) | ((v == best_v) & (i < best_i))
        best_v = jnp.where(take, v, best_v)
        best_i = jnp.where(take, i, best_i)
    return best_v, best_i


def _make_sc_route(n_tokens):
    info = plsc.get_sparse_core_info()
    nw = info.num_cores * info.num_subcores
    lanes = info.num_lanes
    tpw = n_tokens // nw
    nb = tpw // lanes
    sw = _STATS_W
    ne = 16

    mesh = plsc.VectorSubcoreMesh(core_axis_name="c", subcore_axis_name="s")

    @functools.partial(
        pl.kernel, mesh=mesh,
        compiler_params=pltpu.CompilerParams(needs_layout_passes=False),
        out_type=jax.ShapeDtypeStruct((n_tokens, ne), jnp.float32),
        scratch_types=[
            pltpu.VMEM((tpw, sw), jnp.float32),
            pltpu.VMEM((tpw, ne), jnp.float32),
        ],
    )
    def _sc_route(stats_hbm, out_hbm, st_v, out_v):
        wid = lax.axis_index("s") * info.num_cores + lax.axis_index("c")
        base = wid * tpw
        pltpu.sync_copy(stats_hbm.at[pl.ds(base, tpw)], st_v)

        zero16 = jnp.zeros((lanes,), jnp.float32)

        def zbody(j, carry):
            out_v[j] = zero16
            return carry

        lax.fori_loop(0, tpw, zbody, 0)

        iot = lax.iota(jnp.int32, lanes)
        quarter = jnp.full((lanes,), 0.25, jnp.float32)
        neg1 = jnp.full((lanes,), -1.0, jnp.float32)

        def batch(b, carry):
            toks = b * lanes + iot

            def col(off):
                if isinstance(off, int):
                    off = jnp.full((lanes,), off, jnp.int32)
                return plsc.load_gather(st_v, [toks, off])

            g = [col(r) for r in range(4)]
            m = jnp.maximum(jnp.maximum(g[0], g[1]), jnp.maximum(g[2], g[3]))
            eg = [jnp.exp(v - m) for v in g]
            s = eg[0] + eg[1] + eg[2] + eg[3]
            gp = [v / s for v in eg]
            gidx = [jnp.full((lanes,), r, jnp.int32) for r in range(4)]

            big_i = jnp.full((lanes,), 64, jnp.int32)
            p0, gid0 = _argmax_tb(gp, gidx, neg1, big_i)
            gp1 = [jnp.where(gidx[r] == gid0, -1.0, gp[r]) for r in range(4)]
            p1, gid1 = _argmax_tb(gp1, gidx, neg1, big_i)

            conf = col(20)
            blend = (1.0 - conf) * quarter

            cand_v, cand_i = [], []
            for pg, gid in ((p0, gid0), (p1, gid1)):
                eoff = 4 + gid * 4
                lj = [col(eoff + j) for j in range(4)]
                mm = jnp.maximum(jnp.maximum(lj[0], lj[1]),
                                 jnp.maximum(lj[2], lj[3]))
                ej = [jnp.exp(v - mm) for v in lj]
                es = ej[0] + ej[1] + ej[2] + ej[3]
                pj = [v / es * conf + blend for v in ej]
                jdx = [jnp.full((lanes,), j, jnp.int32) for j in range(4)]
                t0, i0 = _argmax_tb(pj, jdx, neg1, big_i)
                pj1 = [jnp.where(jdx[j] == i0, -1.0, pj[j]) for j in range(4)]
                t1, i1 = _argmax_tb(pj1, jdx, neg1, big_i)
                cand_v += [t0 * pg, t1 * pg]
                cand_i += [gid * 4 + i0, gid * 4 + i1]

            tot = cand_v[0] + cand_v[1] + cand_v[2] + cand_v[3] + 1e-9
            cand_v = [v / tot for v in cand_v]

            d0, j0 = _argmax_tb(cand_v, cand_i, neg1, big_i)
            cv1 = [jnp.where(cand_i[k] == j0, -1.0, cand_v[k])
                   for k in range(4)]
            d1, j1 = _argmax_tb(cv1, cand_i, neg1, big_i)

            plsc.store_scatter(out_v, [toks, j0], d0)
            plsc.store_scatter(out_v, [toks, j1], d1)
            return carry

        lax.fori_loop(0, nb, batch, 0)
        pltpu.sync_copy(out_v, out_hbm.at[pl.ds(base, tpw)])

    return _sc_route


def kernel(x, Wg, bg, We, be, Wc1, bc1, Wc2, bc2, priority):
    b, s, h = x.shape
    n = b * s
    g, epg, _ = We.shape
    e = g * epg
    x_flat = x.reshape(n, h)

    pad = _STATS_W - 4 - e
    w24 = jnp.concatenate(
        [Wg, We.reshape(e, h), jnp.zeros((pad, h), jnp.float32)], axis=0)
    b24 = jnp.concatenate(
        [bg, be.reshape(e), jnp.zeros((pad,), jnp.float32)]
    ).reshape(1, _STATS_W)
    p24 = jnp.concatenate(
        [jnp.ones((4,), jnp.float32), priority,
         jnp.ones((pad,), jnp.float32)]).reshape(1, _STATS_W)

    wc2p = jnp.zeros((_STATS_W, Wc1.shape[0]), jnp.float32).at[20].set(Wc2[0])
    bc2row = jnp.broadcast_to(bc2.reshape(1, 1), (1, _STATS_W))
    stats = _tc_stats(x_flat.astype(jnp.bfloat16),
                      w24.T.astype(jnp.bfloat16), b24, p24,
                      Wc1.T.astype(jnp.bfloat16), bc1.reshape(1, -1),
                      wc2p.T.astype(jnp.bfloat16), bc2row)
    disp_flat = _make_sc_route(n)(stats)
    disp = disp_flat.reshape(b, s, e)
    return disp, disp, jnp.zeros((), jnp.float32)
